# baseline (speedup 1.0000x reference)
import jax
import jax.numpy as jnp
from jax.experimental import pallas as pl
from jax.experimental.pallas import tpu as pltpu

ZW = 64
H1 = 256
H2 = 1024
S = 64
IMG = 128
OUT = 3 * S * S
COLT = 1024
NT = OUT // COLT


def _mlp_body(z_ref, w1_ref, b1_ref, w2_ref, b2_ref, w3_ref, b3_ref,
              out_ref, h2_ref):
    t = pl.program_id(0)

    @pl.when(t == 0)
    def _():
        h1 = jax.nn.relu(
            jnp.dot(z_ref[...], w1_ref[...], preferred_element_type=jnp.float32, precision=jax.lax.Precision.HIGHEST)
            + b1_ref[...])
        h2_ref[...] = jax.nn.relu(
            jnp.dot(h1, w2_ref[...], preferred_element_type=jnp.float32, precision=jax.lax.Precision.HIGHEST)
            + b2_ref[...])

    o = jnp.dot(h2_ref[...], w3_ref[...], preferred_element_type=jnp.float32, precision=jax.lax.Precision.HIGHEST)
    out_ref[...] = jax.nn.sigmoid(o + b3_ref[...])


def _stn_body(nobj, n, dec_ref, zw_ref, d_ref, p_ref, out_ref, acc_ref):
    i = pl.program_id(1)
    b = pl.program_id(0)
    row = b * nobj + i

    a = dec_ref[0]

    cx = zw_ref[row, 0] * 2.0 - 1.0
    cy = zw_ref[row, 1] * 2.0 - 1.0
    ww = jnp.maximum(zw_ref[row, 2], 1e-2)
    hh = jnp.maximum(zw_ref[row, 3], 1e-2)

    q = jax.lax.broadcasted_iota(jnp.int32, (1, IMG), 1).astype(jnp.float32)
    gx = (q + 0.5) / (IMG / 2.0) - 1.0
    u = ((gx - cx) / ww + 1.0) * (S / 2.0) - 0.5
    u0 = jnp.floor(u)
    du = u - u0
    t0 = jnp.where((u0 >= 0.0) & (u0 <= S - 1.0), 1.0 - du, 0.0)
    t1 = jnp.where((u0 + 1.0 >= 0.0) & (u0 + 1.0 <= S - 1.0), du, 0.0)
    l = jax.lax.broadcasted_iota(jnp.int32, (IMG, 1), 0).astype(jnp.float32)

    def rx(xsrc, lane_mask):
        m0 = jnp.where((xsrc == u0) & lane_mask, t0, 0.0)
        m1 = jnp.where((xsrc == u0 + 1.0) & lane_mask, t1, 0.0)
        return m0 + m1

    rx0 = rx(l, l < S)
    rx1 = rx(l - S, l >= S)

    b0 = jnp.dot(a, rx0, preferred_element_type=jnp.float32, precision=jax.lax.Precision.HIGHEST)
    b1 = jnp.dot(a, rx1, preferred_element_type=jnp.float32, precision=jax.lax.Precision.HIGHEST)

    pp = jax.lax.broadcasted_iota(jnp.int32, (IMG, 1), 0).astype(jnp.float32)
    gy = (pp + 0.5) / (IMG / 2.0) - 1.0
    v = ((gy - cy) / hh + 1.0) * (S / 2.0) - 0.5
    v0 = jnp.floor(v)
    dv = v - v0
    s0 = jnp.where((v0 >= 0.0) & (v0 <= S - 1.0), 1.0 - dv, 0.0)
    s1 = jnp.where((v0 + 1.0 >= 0.0) & (v0 + 1.0 <= S - 1.0), dv, 0.0)
    jj = jax.lax.broadcasted_iota(jnp.int32, (1, 32), 1).astype(jnp.float32)

    def ry_core(ysrc):
        m0 = jnp.where(ysrc == v0, s0, 0.0)
        m1 = jnp.where(ysrc == v0 + 1.0, s1, 0.0)
        return m0 + m1

    c0 = ry_core(2.0 * jj)
    c1 = ry_core(2.0 * jj + 1.0)

    canvas = jnp.concatenate(
        [jnp.dot(c0, b0[32 * c:32 * (c + 1), :],
                 preferred_element_type=jnp.float32, precision=jax.lax.Precision.HIGHEST)
         + jnp.dot(c1, b1[32 * c:32 * (c + 1), :],
                   preferred_element_type=jnp.float32, precision=jax.lax.Precision.HIGHEST)
         for c in range(3)], axis=0)

    dvec = d_ref[0]
    pvec = p_ref[0]
    deff = jnp.where(pvec == 1.0, dvec, -1e30)
    e = jnp.exp(deff - jnp.max(deff))
    wv = e / jnp.sum(e)
    sel = jax.lax.broadcasted_iota(jnp.int32, (1, n), 1) == i
    wgt = jnp.sum(jnp.where(sel, wv, 0.0))

    contrib = wgt * canvas

    @pl.when(i == 0)
    def _():
        acc_ref[...] = contrib

    @pl.when(jnp.logical_and(i > 0, i < nobj - 1))
    def _():
        acc_ref[...] += contrib

    @pl.when(i == nobj - 1)
    def _():
        merged = acc_ref[...]
        mask = jnp.where(merged < 0.001, 1.0, 0.0)
        out_ref[0] = merged + canvas * mask


def kernel(z_what, z_where, z_present, z_depth, W1, b1, W2, b2, W3, b3):
    B, nobj, _ = z_what.shape
    n = nobj - 1
    M = B * nobj

    z = z_what.reshape(M, ZW)
    bg = jnp.broadcast_to(jnp.array([0.5, 0.5, 1.0, 1.0], jnp.float32),
                          (B, 1, 4))
    zw = jnp.concatenate([z_where, bg], axis=1).reshape(M, 4)
    d = z_depth.reshape(B, 1, n)
    p = z_present.reshape(B, 1, n)

    decoded = pl.pallas_call(
        _mlp_body,
        grid=(NT,),
        in_specs=[
            pl.BlockSpec((M, ZW), lambda t: (0, 0)),
            pl.BlockSpec((ZW, H1), lambda t: (0, 0)),
            pl.BlockSpec((1, H1), lambda t: (0, 0)),
            pl.BlockSpec((H1, H2), lambda t: (0, 0)),
            pl.BlockSpec((1, H2), lambda t: (0, 0)),
            pl.BlockSpec((H2, COLT), lambda t: (0, t)),
            pl.BlockSpec((1, COLT), lambda t: (0, t)),
        ],
        out_specs=pl.BlockSpec((M, COLT), lambda t: (0, t)),
        out_shape=jax.ShapeDtypeStruct((M, OUT), jnp.float32),
        scratch_shapes=[pltpu.VMEM((M, H2), jnp.float32)],
        compiler_params=pltpu.CompilerParams(
            dimension_semantics=("arbitrary",)),
    )(z, W1, b1.reshape(1, H1), W2, b2.reshape(1, H2), W3,
      b3.reshape(1, OUT))

    dec3 = decoded.reshape(M, OUT // 128, 128)

    import functools
    body = functools.partial(_stn_body, nobj, n)
    out = pl.pallas_call(
        body,
        grid=(B, nobj),
        in_specs=[
            pl.BlockSpec((1, OUT // 128, 128), lambda b, i: (b * nobj + i, 0, 0)),
            pl.BlockSpec(memory_space=pltpu.SMEM),
            pl.BlockSpec((1, 1, n), lambda b, i: (b, 0, 0)),
            pl.BlockSpec((1, 1, n), lambda b, i: (b, 0, 0)),
        ],
        out_specs=pl.BlockSpec((1, 3 * IMG, IMG), lambda b, i: (b, 0, 0)),
        out_shape=jax.ShapeDtypeStruct((B, 3 * IMG, IMG), jnp.float32),
        scratch_shapes=[pltpu.VMEM((3 * IMG, IMG), jnp.float32)],
        compiler_params=pltpu.CompilerParams(
            dimension_semantics=("arbitrary", "arbitrary")),
    )(dec3, zw, d, p)

    return out.reshape(B, 3, IMG, IMG)


# speedup vs baseline: 1.1480x; 1.1480x over previous
import functools

import jax
import jax.numpy as jnp
from jax.experimental import pallas as pl
from jax.experimental.pallas import tpu as pltpu

ZW = 64
H1 = 256
H2 = 1024
S = 64
IMG = 128
OUT = 3 * S * S
COLT = 1024
NT = OUT // COLT

_HI = jax.lax.Precision.HIGHEST


def _mlp_body(z_ref, w1_ref, b1_ref, w2_ref, b2_ref, w3h_ref, w3l_ref,
              b3_ref, out_ref, h2h_ref, h2l_ref):
    t = pl.program_id(0)

    @pl.when(t == 0)
    def _():
        h1 = jax.nn.relu(
            jnp.dot(z_ref[...], w1_ref[...],
                    preferred_element_type=jnp.float32, precision=_HI)
            + b1_ref[...])
        h2 = jax.nn.relu(
            jnp.dot(h1, w2_ref[...],
                    preferred_element_type=jnp.float32, precision=_HI)
            + b2_ref[...])
        h2h = h2.astype(jnp.bfloat16)
        h2h_ref[...] = h2h
        h2l_ref[...] = (h2 - h2h.astype(jnp.float32)).astype(jnp.bfloat16)

    h2h = h2h_ref[...]
    o = (jnp.dot(h2h, w3h_ref[...], preferred_element_type=jnp.float32)
         + jnp.dot(h2h, w3l_ref[...], preferred_element_type=jnp.float32)
         + jnp.dot(h2l_ref[...], w3h_ref[...],
                   preferred_element_type=jnp.float32))
    out_ref[...] = jax.nn.sigmoid(o + b3_ref[...])


def _stn_body(nobj, n, dec_ref, zw_ref, d_ref, p_ref, out_ref, acc_ref):
    i = pl.program_id(1)
    b = pl.program_id(0)
    row = b * nobj + i

    a = dec_ref[0]

    cx = zw_ref[row, 0] * 2.0 - 1.0
    cy = zw_ref[row, 1] * 2.0 - 1.0
    ww = jnp.maximum(zw_ref[row, 2], 1e-2)
    hh = jnp.maximum(zw_ref[row, 3], 1e-2)

    q = jax.lax.broadcasted_iota(jnp.int32, (1, IMG), 1).astype(jnp.float32)
    gx = (q + 0.5) / (IMG / 2.0) - 1.0
    u = ((gx - cx) / ww + 1.0) * (S / 2.0) - 0.5
    u0 = jnp.floor(u)
    du = u - u0
    t0 = jnp.where((u0 >= 0.0) & (u0 <= S - 1.0), 1.0 - du, 0.0)
    t1 = jnp.where((u0 + 1.0 >= 0.0) & (u0 + 1.0 <= S - 1.0), du, 0.0)
    ix0 = jnp.clip(u0, 0.0, S - 1.0).astype(jnp.int32)
    ix1 = jnp.clip(u0 + 1.0, 0.0, S - 1.0).astype(jnp.int32)

    ix0b = jnp.broadcast_to(ix0, (96, IMG))
    ix1b = jnp.broadcast_to(ix1, (96, IMG))
    a00 = jnp.take_along_axis(a, ix0b, axis=1)
    a01 = jnp.take_along_axis(a, ix1b, axis=1)
    a10 = jnp.take_along_axis(a, ix0b + S, axis=1)
    a11 = jnp.take_along_axis(a, ix1b + S, axis=1)
    b0 = t0 * a00 + t1 * a01
    b1 = t0 * a10 + t1 * a11
    b0t = b0.T
    b1t = b1.T

    pp = jax.lax.broadcasted_iota(jnp.int32, (1, IMG), 1).astype(jnp.float32)
    gy = (pp + 0.5) / (IMG / 2.0) - 1.0
    v = ((gy - cy) / hh + 1.0) * (S / 2.0) - 0.5
    v0 = jnp.floor(v)
    dv = v - v0
    s0 = jnp.where((v0 >= 0.0) & (v0 <= S - 1.0), 1.0 - dv, 0.0)
    s1 = jnp.where((v0 + 1.0 >= 0.0) & (v0 + 1.0 <= S - 1.0), dv, 0.0)
    jy0 = jnp.clip(v0, 0.0, S - 1.0).astype(jnp.int32)
    jy1 = jnp.clip(v0 + 1.0, 0.0, S - 1.0).astype(jnp.int32)
    r0 = jy0 // 2
    r1 = jy1 // 2
    odd0 = (jy0 - 2 * r0) == 1
    odd1 = (jy1 - 2 * r1) == 1

    chans = []
    for c in range(3):
        idx0 = jnp.broadcast_to(32 * c + r0, (IMG, IMG))
        idx1 = jnp.broadcast_to(32 * c + r1, (IMG, IMG))
        val0 = jnp.where(odd0,
                         jnp.take_along_axis(b1t, idx0, axis=1),
                         jnp.take_along_axis(b0t, idx0, axis=1))
        val1 = jnp.where(odd1,
                         jnp.take_along_axis(b1t, idx1, axis=1),
                         jnp.take_along_axis(b0t, idx1, axis=1))
        chans.append(s0 * val0 + s1 * val1)
    canvas = jnp.concatenate(chans, axis=0)

    dvec = d_ref[0]
    pvec = p_ref[0]
    deff = jnp.where(pvec == 1.0, dvec, -1e30)
    e = jnp.exp(deff - jnp.max(deff))
    wv = e / jnp.sum(e)
    sel = jax.lax.broadcasted_iota(jnp.int32, (1, n), 1) == i
    wgt = jnp.sum(jnp.where(sel, wv, 0.0))

    contrib = wgt * canvas

    @pl.when(i == 0)
    def _():
        acc_ref[...] = contrib

    @pl.when(jnp.logical_and(i > 0, i < nobj - 1))
    def _():
        acc_ref[...] += contrib

    @pl.when(i == nobj - 1)
    def _():
        merged = acc_ref[...]
        mask = jnp.where(merged < 0.001, 1.0, 0.0)
        out_ref[0] = merged + canvas * mask


def kernel(z_what, z_where, z_present, z_depth, W1, b1, W2, b2, W3, b3):
    B, nobj, _ = z_what.shape
    n = nobj - 1
    M = B * nobj

    z = z_what.reshape(M, ZW)
    bg = jnp.broadcast_to(jnp.array([0.5, 0.5, 1.0, 1.0], jnp.float32),
                          (B, 1, 4))
    zw = jnp.concatenate([z_where, bg], axis=1).reshape(M, 4)
    d = z_depth.reshape(B, 1, n)
    p = z_present.reshape(B, 1, n)
    w3h = W3.astype(jnp.bfloat16)
    w3l = (W3 - w3h.astype(jnp.float32)).astype(jnp.bfloat16)

    decoded = pl.pallas_call(
        _mlp_body,
        grid=(NT,),
        in_specs=[
            pl.BlockSpec((M, ZW), lambda t: (0, 0)),
            pl.BlockSpec((ZW, H1), lambda t: (0, 0)),
            pl.BlockSpec((1, H1), lambda t: (0, 0)),
            pl.BlockSpec((H1, H2), lambda t: (0, 0)),
            pl.BlockSpec((1, H2), lambda t: (0, 0)),
            pl.BlockSpec((H2, COLT), lambda t: (0, t)),
            pl.BlockSpec((H2, COLT), lambda t: (0, t)),
            pl.BlockSpec((1, COLT), lambda t: (0, t)),
        ],
        out_specs=pl.BlockSpec((M, COLT), lambda t: (0, t)),
        out_shape=jax.ShapeDtypeStruct((M, OUT), jnp.float32),
        scratch_shapes=[pltpu.VMEM((M, H2), jnp.bfloat16),
                        pltpu.VMEM((M, H2), jnp.bfloat16)],
        compiler_params=pltpu.CompilerParams(
            dimension_semantics=("arbitrary",)),
    )(z, W1, b1.reshape(1, H1), W2, b2.reshape(1, H2), w3h, w3l,
      b3.reshape(1, OUT))

    dec3 = decoded.reshape(M, OUT // 128, 128)

    body = functools.partial(_stn_body, nobj, n)
    out = pl.pallas_call(
        body,
        grid=(B, nobj),
        in_specs=[
            pl.BlockSpec((1, OUT // 128, 128), lambda b, i: (b * nobj + i, 0, 0)),
            pl.BlockSpec(memory_space=pltpu.SMEM),
            pl.BlockSpec((1, 1, n), lambda b, i: (b, 0, 0)),
            pl.BlockSpec((1, 1, n), lambda b, i: (b, 0, 0)),
        ],
        out_specs=pl.BlockSpec((1, 3 * IMG, IMG), lambda b, i: (b, 0, 0)),
        out_shape=jax.ShapeDtypeStruct((B, 3 * IMG, IMG), jnp.float32),
        scratch_shapes=[pltpu.VMEM((3 * IMG, IMG), jnp.float32)],
        compiler_params=pltpu.CompilerParams(
            dimension_semantics=("arbitrary", "arbitrary")),
    )(dec3, zw, d, p)

    return jnp.swapaxes(out.reshape(B, 3, IMG, IMG), 2, 3)


# speedup vs baseline: 1.9553x; 1.7032x over previous
import functools

import jax
import jax.numpy as jnp
from jax.experimental import pallas as pl
from jax.experimental.pallas import tpu as pltpu

ZW = 64
H1 = 256
H2 = 1024
S = 64
IMG = 128
OUT = 3 * S * S
COLT = 1024
NT = OUT // COLT

_HI = jax.lax.Precision.HIGHEST


def _mlp_body(z_ref, w1_ref, b1_ref, w2_ref, b2_ref, w3h_ref, w3l_ref,
              b3_ref, out_ref, h2h_ref, h2l_ref):
    t = pl.program_id(0)

    @pl.when(t == 0)
    def _():
        h1 = jax.nn.relu(
            jnp.dot(z_ref[...], w1_ref[...],
                    preferred_element_type=jnp.float32, precision=_HI)
            + b1_ref[...])
        h2 = jax.nn.relu(
            jnp.dot(h1, w2_ref[...],
                    preferred_element_type=jnp.float32, precision=_HI)
            + b2_ref[...])
        h2h = h2.astype(jnp.bfloat16)
        h2h_ref[...] = h2h
        h2l_ref[...] = (h2 - h2h.astype(jnp.float32)).astype(jnp.bfloat16)

    h2h = h2h_ref[...]
    o = (jnp.dot(h2h, w3h_ref[...], preferred_element_type=jnp.float32)
         + jnp.dot(h2h, w3l_ref[...], preferred_element_type=jnp.float32)
         + jnp.dot(h2l_ref[...], w3h_ref[...],
                   preferred_element_type=jnp.float32))
    out_ref[...] = jax.nn.sigmoid(o + b3_ref[...])


def _stn_one(a, cx, cy, ww, hh):
    q = jax.lax.broadcasted_iota(jnp.int32, (1, IMG), 1).astype(jnp.float32)
    gx = (q + 0.5) / (IMG / 2.0) - 1.0
    u = ((gx - cx) / ww + 1.0) * (S / 2.0) - 0.5
    u0 = jnp.floor(u)
    du = u - u0
    t0 = jnp.where((u0 >= 0.0) & (u0 <= S - 1.0), 1.0 - du, 0.0)
    t1 = jnp.where((u0 + 1.0 >= 0.0) & (u0 + 1.0 <= S - 1.0), du, 0.0)
    ix0 = jnp.clip(u0, 0.0, S - 1.0).astype(jnp.int32)
    ix1 = jnp.clip(u0 + 1.0, 0.0, S - 1.0).astype(jnp.int32)

    ix0b = jnp.broadcast_to(ix0, (96, IMG))
    ix1b = jnp.broadcast_to(ix1, (96, IMG))
    a00 = jnp.take_along_axis(a, ix0b, axis=1)
    a01 = jnp.take_along_axis(a, ix1b, axis=1)
    a10 = jnp.take_along_axis(a, ix0b + S, axis=1)
    a11 = jnp.take_along_axis(a, ix1b + S, axis=1)
    b0 = t0 * a00 + t1 * a01
    b1 = t0 * a10 + t1 * a11

    pp = jax.lax.broadcasted_iota(jnp.int32, (IMG, 1), 0).astype(jnp.float32)
    gy = (pp + 0.5) / (IMG / 2.0) - 1.0
    v = ((gy - cy) / hh + 1.0) * (S / 2.0) - 0.5
    v0 = jnp.floor(v)
    dv = v - v0
    s0 = jnp.where((v0 >= 0.0) & (v0 <= S - 1.0), 1.0 - dv, 0.0)
    s1 = jnp.where((v0 + 1.0 >= 0.0) & (v0 + 1.0 <= S - 1.0), dv, 0.0)
    jj = jax.lax.broadcasted_iota(jnp.int32, (1, 32), 1).astype(jnp.float32)

    def ry_core(ysrc):
        m0 = jnp.where(ysrc == v0, s0, 0.0)
        m1 = jnp.where(ysrc == v0 + 1.0, s1, 0.0)
        return m0 + m1

    c0 = ry_core(2.0 * jj)
    c1 = ry_core(2.0 * jj + 1.0)

    return jnp.concatenate(
        [jnp.dot(c0, b0[32 * c:32 * (c + 1), :],
                 preferred_element_type=jnp.float32, precision=_HI)
         + jnp.dot(c1, b1[32 * c:32 * (c + 1), :],
                   preferred_element_type=jnp.float32, precision=_HI)
         for c in range(3)], axis=0)


def _stn_body(nobj, n, ob, dec_ref, zw_ref, d_ref, p_ref, out_ref, acc_ref):
    ip = pl.program_id(1)
    b = pl.program_id(0)
    nstep = nobj // ob
    ibase = ip * ob

    dvec = d_ref[0]
    pvec = p_ref[0]
    deff = jnp.where(pvec == 1.0, dvec, -1e30)
    e = jnp.exp(deff - jnp.max(deff))
    wv = e / jnp.sum(e)
    obj_iota = jax.lax.broadcasted_iota(jnp.int32, (1, n), 1)

    total = None
    canvas = None
    for k in range(ob):
        i = ibase + k
        row = b * nobj + i
        cx = zw_ref[row, 0] * 2.0 - 1.0
        cy = zw_ref[row, 1] * 2.0 - 1.0
        ww = jnp.maximum(zw_ref[row, 2], 1e-2)
        hh = jnp.maximum(zw_ref[row, 3], 1e-2)
        canvas = _stn_one(dec_ref[k], cx, cy, ww, hh)
        wgt = jnp.sum(jnp.where(obj_iota == i, wv, 0.0))
        contrib = wgt * canvas
        total = contrib if total is None else total + contrib

    @pl.when(ip == 0)
    def _():
        acc_ref[...] = total

    @pl.when(jnp.logical_and(ip > 0, ip < nstep - 1))
    def _():
        acc_ref[...] += total

    @pl.when(ip == nstep - 1)
    def _():
        merged = acc_ref[...] + total
        mask = jnp.where(merged < 0.001, 1.0, 0.0)
        out_ref[0] = merged + canvas * mask


def kernel(z_what, z_where, z_present, z_depth, W1, b1, W2, b2, W3, b3):
    B, nobj, _ = z_what.shape
    n = nobj - 1
    M = B * nobj

    z = z_what.reshape(M, ZW)
    bg = jnp.broadcast_to(jnp.array([0.5, 0.5, 1.0, 1.0], jnp.float32),
                          (B, 1, 4))
    zw = jnp.concatenate([z_where, bg], axis=1).reshape(M, 4)
    d = z_depth.reshape(B, 1, n)
    p = z_present.reshape(B, 1, n)
    w3h = W3.astype(jnp.bfloat16)
    w3l = (W3 - w3h.astype(jnp.float32)).astype(jnp.bfloat16)

    decoded = pl.pallas_call(
        _mlp_body,
        grid=(NT,),
        in_specs=[
            pl.BlockSpec((M, ZW), lambda t: (0, 0)),
            pl.BlockSpec((ZW, H1), lambda t: (0, 0)),
            pl.BlockSpec((1, H1), lambda t: (0, 0)),
            pl.BlockSpec((H1, H2), lambda t: (0, 0)),
            pl.BlockSpec((1, H2), lambda t: (0, 0)),
            pl.BlockSpec((H2, COLT), lambda t: (0, t)),
            pl.BlockSpec((H2, COLT), lambda t: (0, t)),
            pl.BlockSpec((1, COLT), lambda t: (0, t)),
        ],
        out_specs=pl.BlockSpec((M, COLT), lambda t: (0, t)),
        out_shape=jax.ShapeDtypeStruct((M, OUT), jnp.float32),
        scratch_shapes=[pltpu.VMEM((M, H2), jnp.bfloat16),
                        pltpu.VMEM((M, H2), jnp.bfloat16)],
        compiler_params=pltpu.CompilerParams(
            dimension_semantics=("arbitrary",)),
    )(z, W1, b1.reshape(1, H1), W2, b2.reshape(1, H2), w3h, w3l,
      b3.reshape(1, OUT))

    dec3 = decoded.reshape(M, OUT // 128, 128)

    ob = 4 if nobj % 4 == 0 else 1
    nstep = nobj // ob
    body = functools.partial(_stn_body, nobj, n, ob)
    out = pl.pallas_call(
        body,
        grid=(B, nstep),
        in_specs=[
            pl.BlockSpec((ob, OUT // 128, 128),
                         lambda b, i: (b * nstep + i, 0, 0)),
            pl.BlockSpec(memory_space=pltpu.SMEM),
            pl.BlockSpec((1, 1, n), lambda b, i: (b, 0, 0)),
            pl.BlockSpec((1, 1, n), lambda b, i: (b, 0, 0)),
        ],
        out_specs=pl.BlockSpec((1, 3 * IMG, IMG), lambda b, i: (b, 0, 0)),
        out_shape=jax.ShapeDtypeStruct((B, 3 * IMG, IMG), jnp.float32),
        scratch_shapes=[pltpu.VMEM((3 * IMG, IMG), jnp.float32)],
        compiler_params=pltpu.CompilerParams(
            dimension_semantics=("arbitrary", "arbitrary")),
    )(dec3, zw, d, p)

    return out.reshape(B, 3, IMG, IMG)
